# baseline (speedup 1.0000x reference)
import functools

import jax
import jax.numpy as jnp
from jax import lax
from jax.experimental import pallas as pl
from jax.experimental.pallas import tpu as pltpu
from jax.experimental.pallas import tpu_sc as plsc

N = 10000
E = 160000
E_LG = 640000
F = 128
OUT_FEATS = 8
EPS = 1e-5

NC = 2
NS = 16
NW = NC * NS

NP = 10240
EP = 163840

def _mesh():
    return plsc.VectorSubcoreMesh(core_axis_name="c", subcore_axis_name="s",
                                  num_cores=NC, num_subcores=NS)


_SC_PARAMS = pltpu.CompilerParams(needs_layout_passes=False)


def _vfill(ref, nwords, value, dtype):
    vec = jnp.full((16,), value, dtype)

    def body(i, carry):
        ref[pl.ds(i * 16, 16)] = vec
        return carry

    lax.fori_loop(0, nwords // 16, body, 0)


def _vfill2d(ref, nrows, value, dtype):
    vec = jnp.full((16,), value, dtype)

    def body(r, carry):
        for kk in range(F // 16):
            ref[r, pl.ds(kk * 16, 16)] = vec
        return carry

    lax.fori_loop(0, nrows, body, 0)


def _sc_count(idx, SP, B):
    (M,) = idx.shape
    PW = M // NW
    NB = PW // B
    S16 = SP // 16
    DB = 2048 if S16 % 2048 == 0 else S16
    ND = S16 // DB
    B16 = ((B + 15) // 16) * 16

    @functools.partial(
        pl.kernel,
        out_type=jax.ShapeDtypeStruct((2 * SP,), jnp.float32),
        mesh=_mesh(),
        compiler_params=_SC_PARAMS,
        scratch_types=[
            pltpu.VMEM((B,), jnp.int32),
            pltpu.VMEM((B16,), jnp.float32),
            pltpu.VMEM((DB,), jnp.float32),
            pltpu.VMEM_SHARED((SP,), jnp.float32),
        ],
    )
    def k(idx_hbm, out_hbm, istage, ones_v, zbuf, acc):
        c = lax.axis_index("c")
        s = lax.axis_index("s")
        wid = c * NS + s
        _vfill(ones_v, B16, 1.0, jnp.float32)
        _vfill(zbuf, DB, 0.0, jnp.float32)
        for dd in range(ND):
            pltpu.sync_copy(zbuf, acc.at[pl.ds(s * S16 + dd * DB, DB)])
        plsc.subcore_barrier()

        def blk(b, carry):
            pltpu.sync_copy(idx_hbm.at[pl.ds(wid * PW + b * B, B)], istage)
            pltpu.sync_copy(ones_v.at[pl.ds(0, B)], acc.at[istage], add=True)
            return carry

        lax.fori_loop(0, NB, blk, 0)
        plsc.subcore_barrier()
        for dd in range(ND):
            pltpu.sync_copy(acc.at[pl.ds(s * S16 + dd * DB, DB)],
                            out_hbm.at[pl.ds(c * SP + s * S16 + dd * DB, DB)])

    return k(idx)


def _sc_segsum_nodes(table, src, dst, with_rows=False, linear=False):
    (M,) = dst.shape
    PW = M // NW
    B = 200
    NB = PW // B
    R16 = NP // 16

    out_type = [jax.ShapeDtypeStruct((2 * NP, F), jnp.float32)]
    if with_rows:
        out_type.append(jax.ShapeDtypeStruct((M, F), jnp.float32))

    scratch = [
        pltpu.VMEM((B,), jnp.int32),
        pltpu.VMEM((B,), jnp.int32),
        pltpu.VMEM((B, F), jnp.float32),
        pltpu.VMEM((128, F), jnp.float32),
        pltpu.SemaphoreType.DMA,
        pltpu.VMEM_SHARED((NP, F), jnp.float32),
    ]

    @functools.partial(pl.kernel, out_type=out_type, mesh=_mesh(),
                       compiler_params=_SC_PARAMS, scratch_types=scratch)
    def k(table_hbm, src_hbm, dst_hbm, *refs):
        if with_rows:
            out_hbm, rows_hbm = refs[0], refs[1]
            sstage, dstage, rows, zbuf, sem, acc = refs[2:]
        else:
            out_hbm = refs[0]
            sstage, dstage, rows, zbuf, sem, acc = refs[1:]
        c = lax.axis_index("c")
        s = lax.axis_index("s")
        wid = c * NS + s
        _vfill2d(zbuf, 128, 0.0, jnp.float32)
        for kk in range(R16 // 128):
            pltpu.sync_copy(zbuf, acc.at[pl.ds(s * R16 + kk * 128, 128)])
        plsc.subcore_barrier()

        def blk(b, carry):
            base = wid * PW + b * B
            pltpu.sync_copy(dst_hbm.at[pl.ds(base, B)], dstage)
            if linear:
                pltpu.sync_copy(table_hbm.at[pl.ds(base, B)], rows)
            else:
                pltpu.sync_copy(src_hbm.at[pl.ds(base, B)], sstage)
                pltpu.async_copy(table_hbm.at[sstage], rows, sem).wait()
            if with_rows:
                pltpu.sync_copy(rows, rows_hbm.at[pl.ds(base, B)])
            pltpu.sync_copy(rows, acc.at[dstage], add=True)
            return carry

        lax.fori_loop(0, NB, blk, 0)
        plsc.subcore_barrier()
        pltpu.sync_copy(acc.at[pl.ds(s * R16, R16)],
                        out_hbm.at[pl.ds(c * NP + s * R16, R16)])

    res = k(table, src, dst)
    if with_rows:
        return tuple(res)
    res = jax.tree.leaves(res)
    return (res[0],)


CH = 10000
NCH_SC = 8
NCH_TOT = NC * NCH_SC
ACC_ROWS = 10240
SB = 2000
GB = 128
FB = 2048
CAPR = 40960
BINSZ = NS * NCH_TOT * CAPR
BUFW = 4352


def _sc_bin_lg(lsrc, ldst):
    scratch = ([pltpu.VMEM((SB,), jnp.int32), pltpu.VMEM((SB,), jnp.int32)]
               + [pltpu.VMEM((BUFW,), jnp.int32) for _ in range(2 * NCH_SC)]
               + [pltpu.VMEM((16,), jnp.int32)])

    @functools.partial(
        pl.kernel,
        out_type=[jax.ShapeDtypeStruct((BINSZ,), jnp.int32),
                  jax.ShapeDtypeStruct((BINSZ,), jnp.int32),
                  jax.ShapeDtypeStruct((NC * NS * 16,), jnp.int32)],
        mesh=_mesh(),
        compiler_params=_SC_PARAMS,
        scratch_types=scratch,
    )
    def k(lsrc_hbm, ldst_hbm, bls_hbm, bld_hbm, cnt_hbm,
          ls_st, ld_st, *rest):
        bls = list(rest[:NCH_SC])
        bld = list(rest[NCH_SC:2 * NCH_SC])
        cvm = rest[2 * NCH_SC]
        c = lax.axis_index("c")
        s = lax.axis_index("s")
        PW = E_LG // NS
        dump_ls = jnp.zeros((16,), jnp.int32)
        dump_ld = jnp.full((16,), CH, jnp.int32)

        def flush_one(ci, off, fill):
            reg = (s * NCH_TOT + c * NCH_SC + ci) * CAPR
            pos = pl.multiple_of(reg + off, 256)
            pltpu.sync_copy(bls[ci].at[pl.ds(0, FB)],
                            bls_hbm.at[pl.ds(pos, FB)])
            pltpu.sync_copy(bld[ci].at[pl.ds(0, FB)],
                            bld_hbm.at[pl.ds(pos, FB)])

            def shift(kk, carry, ci=ci):
                t1 = bls[ci][pl.ds(FB + kk * 16, 16)]
                t2 = bld[ci][pl.ds(FB + kk * 16, 16)]
                bls[ci][pl.ds(kk * 16, 16)] = t1
                bld[ci][pl.ds(kk * 16, 16)] = t2
                return carry

            lax.fori_loop(0, (fill - FB + 15) // 16, shift, 0)
            return off + FB, fill - FB

        def sblk(b, carry):
            pltpu.sync_copy(lsrc_hbm.at[pl.ds(s * PW + b * SB, SB)], ls_st)
            pltpu.sync_copy(ldst_hbm.at[pl.ds(s * PW + b * SB, SB)], ld_st)

            def vb(v, carry):
                st = list(carry)
                lsv = ls_st[pl.ds(v * 16, 16)]
                ldv = ld_st[pl.ds(v * 16, 16)]
                chv = ldv // CH
                for ci in range(NCH_SC):
                    fill = st[ci]
                    chunk = c * NCH_SC + ci
                    m = chv == chunk
                    plsc.store_compressed(bls[ci].at[pl.ds(fill, 16)], lsv,
                                          mask=m)
                    plsc.store_compressed(bld[ci].at[pl.ds(fill, 16)],
                                          ldv - chunk * CH, mask=m)
                    st[ci] = fill + jnp.sum(jnp.where(m, 1, 0))
                return tuple(st)

            fills = lax.fori_loop(0, SB // 16, vb, carry[:NCH_SC])
            st = list(fills) + list(carry[NCH_SC:])
            for ci in range(NCH_SC):
                off, fill = st[NCH_SC + ci], st[ci]
                off, fill = lax.cond(
                    fill >= FB,
                    lambda o, f, ci=ci: flush_one(ci, o, f),
                    lambda o, f: (o, f), off, fill)
                st[NCH_SC + ci], st[ci] = off, fill
            return tuple(st)

        st = lax.fori_loop(0, PW // SB, sblk, (0,) * (2 * NCH_SC))

        cvec = jnp.zeros((16,), jnp.int32)
        lane = lax.iota(jnp.int32, 16)
        for ci in range(NCH_SC):
            off, fill = st[NCH_SC + ci], st[ci]
            pad = (-(off + fill)) % (2 * GB)
            for kk in range(2 * GB // 16):
                bls[ci][pl.ds(fill + kk * 16, 16)] = dump_ls
                bld[ci][pl.ds(fill + kk * 16, 16)] = dump_ld
            fill = fill + pad
            total = off + fill
            reg = (s * NCH_TOT + c * NCH_SC + ci) * CAPR

            def fblk(j, carry, ci=ci, reg=reg, off=off):
                pos = pl.multiple_of(reg + off + j * 256, 256)
                pltpu.sync_copy(bls[ci].at[pl.ds(j * 256, 256)],
                                bls_hbm.at[pl.ds(pos, 256)])
                pltpu.sync_copy(bld[ci].at[pl.ds(j * 256, 256)],
                                bld_hbm.at[pl.ds(pos, 256)])
                return carry

            lax.fori_loop(0, (fill + 255) // 256, fblk, 0)
            cvec = cvec + jnp.where(lane == c * NCH_SC + ci, total, 0)
        cvm[pl.ds(0, 16)] = cvec
        pltpu.sync_copy(cvm, cnt_hbm.at[pl.ds((c * NS + s) * 16, 16)])

    return k(lsrc, ldst)


def _sc_segsum_lg_stream(y, bls, bld, counts):
    NSTR = 4
    SGB = 64
    scratch = ([pltpu.VMEM((SGB,), jnp.int32) for _ in range(2 * NSTR)]
               + [pltpu.VMEM((SGB, F), jnp.float32) for _ in range(NSTR)]
               + [pltpu.VMEM((64, F), jnp.float32),
                  pltpu.VMEM((16,), jnp.int32)]
               + [pltpu.SemaphoreType.DMA] * (2 * NSTR)
               + [pltpu.VMEM_SHARED((ACC_ROWS, F), jnp.float32)])

    @functools.partial(
        pl.kernel,
        out_type=jax.ShapeDtypeStruct((E, F), jnp.float32),
        mesh=_mesh(),
        compiler_params=_SC_PARAMS,
        scratch_types=scratch,
    )
    def k(y_hbm, bls_hbm, bld_hbm, cnt_hbm, out_hbm, *refs):
        lsv = list(refs[0:NSTR])
        ldv = list(refs[NSTR:2 * NSTR])
        rowsv = list(refs[2 * NSTR:3 * NSTR])
        zbuf = refs[3 * NSTR]
        cvm = refs[3 * NSTR + 1]
        semg = list(refs[3 * NSTR + 2:3 * NSTR + 2 + NSTR])
        sema = list(refs[3 * NSTR + 2 + NSTR:3 * NSTR + 2 + 2 * NSTR])
        acc = refs[-1]
        c = lax.axis_index("c")
        s = lax.axis_index("s")
        _vfill2d(zbuf, 64, 0.0, jnp.float32)
        pltpu.sync_copy(cnt_hbm.at[pl.ds((c * NS + s) * 16, 16)], cvm)
        cvec = cvm[pl.ds(0, 16)]
        lane = lax.iota(jnp.int32, 16)

        for ci in range(NCH_SC):
            chunk = c * NCH_SC + ci
            base_seg = chunk * CH
            reg = (s * NCH_TOT + chunk) * CAPR
            for kk in range(ACC_ROWS // NS // 64):
                pltpu.sync_copy(zbuf, acc.at[pl.ds(s * (ACC_ROWS // NS)
                                                   + kk * 64, 64)])
            plsc.subcore_barrier()
            cnt = jnp.sum(jnp.where(lane == chunk, cvec, 0))
            npair = cnt // (2 * GB)

            def pair(p, carry):
                b0 = reg + p * 2 * GB
                gs = []
                for q in range(NSTR):
                    pltpu.sync_copy(bls_hbm.at[pl.ds(b0 + q * SGB, SGB)],
                                    lsv[q])
                    pltpu.sync_copy(bld_hbm.at[pl.ds(b0 + q * SGB, SGB)],
                                    ldv[q])
                    gs.append(pltpu.async_copy(y_hbm.at[lsv[q]], rowsv[q],
                                               semg[q]))
                aa = []
                for q in range(NSTR):
                    gs[q].wait()
                    aa.append(pltpu.async_copy(rowsv[q], acc.at[ldv[q]],
                                               sema[q], add=True))
                for q in range(NSTR):
                    aa[q].wait()
                return carry

            lax.fori_loop(0, npair, pair, 0)
            plsc.subcore_barrier()

            @pl.when(s < 10)
            def _():
                pltpu.sync_copy(acc.at[pl.ds(s * 1000, 1000)],
                                out_hbm.at[pl.ds(base_seg + s * 1000, 1000)])

            plsc.subcore_barrier()

    return k(y, bls, bld, counts)


def _dot(a, w):
    return lax.dot_general(a, w, (((1,), (0,)), ((), ())),
                           preferred_element_type=jnp.float32)


def _tc_sum2(a, b):
    nrows = a.shape[0]
    RB = 2000
    G = nrows // RB

    def body(a_ref, b_ref, o_ref):
        o_ref[...] = a_ref[...] + b_ref[...]

    rb = lambda i: (i, 0)
    return pl.pallas_call(
        body,
        grid=(G,),
        in_specs=[pl.BlockSpec((RB, F), rb), pl.BlockSpec((RB, F), rb)],
        out_specs=pl.BlockSpec((RB, F), rb),
        out_shape=jax.ShapeDtypeStruct((nrows, F), jnp.float32),
    )(a, b)


def _tc_affine5(x, dg0, dg1, a3, a4, a5, wx, wd, w3, w4, w5, bsum):
    nrows = x.shape[0]
    RB = 2000
    G = nrows // RB

    def body(x_ref, dg0_ref, dg1_ref, a3_ref, a4_ref, a5_ref,
             wx_ref, wd_ref, w3_ref, w4_ref, w5_ref, b_ref,
             out_ref, st_ref, acc):
        i = pl.program_id(0)
        x = x_ref[...]
        deg = dg0_ref[...] + dg1_ref[...]
        t = (_dot(x, wx_ref[...]) + _dot(x * deg, wd_ref[...])
             + _dot(a3_ref[...], w3_ref[...]) + _dot(a4_ref[...], w4_ref[...])
             + _dot(a5_ref[...], w5_ref[...]) + b_ref[...])
        lane = lax.broadcasted_iota(jnp.int32, t.shape, 1)
        t = jnp.where(lane >= F // 2, jnp.maximum(t, 0.0), t)
        out_ref[...] = t

        @pl.when(i == 0)
        def _():
            acc[...] = jnp.zeros_like(acc)

        acc[0:1, :] += jnp.sum(t, axis=0, keepdims=True)
        acc[1:2, :] += jnp.sum(t * t, axis=0, keepdims=True)
        st_ref[...] = acc[...]

    rb = lambda i: (i, 0)
    c0 = lambda i: (0, 0)
    wspec = pl.BlockSpec((F, F), c0)
    return pl.pallas_call(
        body,
        grid=(G,),
        in_specs=[pl.BlockSpec((RB, F), rb),
                  pl.BlockSpec((RB, 1), rb), pl.BlockSpec((RB, 1), rb),
                  pl.BlockSpec((RB, F), rb), pl.BlockSpec((RB, F), rb),
                  pl.BlockSpec((RB, F), rb),
                  wspec, wspec, wspec, wspec, wspec,
                  pl.BlockSpec((1, F), c0)],
        out_specs=[pl.BlockSpec((RB, F), rb), pl.BlockSpec((2, F), c0)],
        out_shape=[jax.ShapeDtypeStruct((nrows, F), jnp.float32),
                   jax.ShapeDtypeStruct((2, F), jnp.float32)],
        scratch_shapes=[pltpu.VMEM((2, F), jnp.float32)],
    )(x, dg0, dg1, a3, a4, a5, wx, wd, w3, w4, w5, bsum)


def _tc_norm(t, stats, scale, bias, wf=None, bf=None):
    nrows = t.shape[0]
    RB = 2000
    G = nrows // RB
    inv_n = 1.0 / nrows
    fuse = wf is not None

    def body(*refs):
        if fuse:
            t_ref, st_ref, sc_ref, bi_ref, wf_ref, bf_ref, o_ref = refs
        else:
            t_ref, st_ref, sc_ref, bi_ref, o_ref = refs
        st = st_ref[...]
        mean = st[0:1, :] * inv_n
        var = st[1:2, :] * inv_n - mean * mean
        inv = lax.rsqrt(var + EPS)
        y = (t_ref[...] - mean) * (inv * sc_ref[...]) + bi_ref[...]
        if fuse:
            o_ref[...] = _dot(y, wf_ref[...]) + bf_ref[...]
        else:
            o_ref[...] = y

    rb = lambda i: (i, 0)
    c0 = lambda i: (0, 0)
    in_specs = [pl.BlockSpec((RB, F), rb), pl.BlockSpec((2, F), c0),
                pl.BlockSpec((1, F), c0), pl.BlockSpec((1, F), c0)]
    args = [t, stats, scale, bias]
    if fuse:
        in_specs += [pl.BlockSpec((F, OUT_FEATS), c0),
                     pl.BlockSpec((1, OUT_FEATS), c0)]
        args += [wf, bf]
        out_spec = pl.BlockSpec((RB, OUT_FEATS), rb)
        out_shape = jax.ShapeDtypeStruct((nrows, OUT_FEATS), jnp.float32)
    else:
        out_spec = pl.BlockSpec((RB, F), rb)
        out_shape = jax.ShapeDtypeStruct((nrows, F), jnp.float32)
    return pl.pallas_call(
        body, grid=(G,), in_specs=in_specs, out_specs=out_spec,
        out_shape=out_shape,
    )(*args)


def _split_partials(p):
    return p[:N], p[NP:NP + N]


def kernel(h, lg_h, edge_index, lg_edge_index, params):
    src, dst = edge_index[0], edge_index[1]
    lsrc, ldst = lg_edge_index[0], lg_edge_index[1]

    cnt_g = _sc_count(dst, NP, 1000)
    cnt_lg = _sc_count(ldst, EP, 2000)
    dg0 = cnt_g[:N].reshape(N, 1)
    dg1 = cnt_g[NP:NP + N].reshape(N, 1)
    dl0 = cnt_lg[:E].reshape(E, 1)
    dl1 = cnt_lg[EP:EP + E].reshape(E, 1)

    p0, p1 = params['modules'][0], params['modules'][1]

    def wmat(p, name):
        return p[name][0]

    z1p, pmpd_x = _sc_segsum_nodes(h, src, dst, with_rows=True)
    z1 = _tc_sum2(*_split_partials(z1p))
    (z2p,) = _sc_segsum_nodes(z1, src, dst)
    z2 = _tc_sum2(*_split_partials(z2p))
    (pyp,) = _sc_segsum_nodes(lg_h, src, dst, linear=True)
    py = _tc_sum2(*_split_partials(pyp))
    t_x, st_x = _tc_affine5(
        h, dg0, dg1, z1, z2, py,
        wmat(p0, 'theta_x'), wmat(p0, 'theta_deg'),
        p0['theta_list'][0][0], p0['theta_list'][1][0], wmat(p0, 'theta_y'),
        (p0['theta_x'][1] + p0['theta_deg'][1] + p0['theta_y'][1]
         + p0['theta_list'][0][1] + p0['theta_list'][1][1]).reshape(1, F))
    xn = _tc_norm(t_x, st_x, p0['bn_x'][0].reshape(1, F),
                  p0['bn_x'][1].reshape(1, F))

    bls, bld, bcnt = _sc_bin_lg(lsrc, ldst)
    w1 = _sc_segsum_lg_stream(lg_h, bls, bld, bcnt)
    w2 = _sc_segsum_lg_stream(w1, bls, bld, bcnt)
    t_y, st_y = _tc_affine5(
        lg_h, dl0, dl1, w1, w2, pmpd_x,
        wmat(p0, 'gamma_y'), wmat(p0, 'gamma_deg'),
        p0['gamma_list'][0][0], p0['gamma_list'][1][0], wmat(p0, 'gamma_x'),
        (p0['gamma_y'][1] + p0['gamma_deg'][1] + p0['gamma_x'][1]
         + p0['gamma_list'][0][1] + p0['gamma_list'][1][1]).reshape(1, F))
    yn = _tc_norm(t_y, st_y, p0['bn_y'][0].reshape(1, F),
                  p0['bn_y'][1].reshape(1, F))

    (z1p2,) = _sc_segsum_nodes(xn, src, dst)
    z1_2 = _tc_sum2(*_split_partials(z1p2))
    (z2p2,) = _sc_segsum_nodes(z1_2, src, dst)
    z2_2 = _tc_sum2(*_split_partials(z2p2))
    (pyp2,) = _sc_segsum_nodes(yn, src, dst, linear=True)
    py2 = _tc_sum2(*_split_partials(pyp2))
    t2, st2 = _tc_affine5(
        xn, dg0, dg1, z1_2, z2_2, py2,
        wmat(p1, 'theta_x'), wmat(p1, 'theta_deg'),
        p1['theta_list'][0][0], p1['theta_list'][1][0], wmat(p1, 'theta_y'),
        (p1['theta_x'][1] + p1['theta_deg'][1] + p1['theta_y'][1]
         + p1['theta_list'][0][1] + p1['theta_list'][1][1]).reshape(1, F))
    wf, bf = params['linear']
    return _tc_norm(t2, st2, p1['bn_x'][0].reshape(1, F),
                    p1['bn_x'][1].reshape(1, F),
                    wf=wf, bf=bf.reshape(1, OUT_FEATS))


# speedup vs baseline: 1.0524x; 1.0524x over previous
import functools

import jax
import jax.numpy as jnp
from jax import lax
from jax.experimental import pallas as pl
from jax.experimental.pallas import tpu as pltpu
from jax.experimental.pallas import tpu_sc as plsc

N = 10000
E = 160000
E_LG = 640000
F = 128
OUT_FEATS = 8
EPS = 1e-5

NC = 2
NS = 16
NW = NC * NS

NP = 10240
EP = 163840

def _mesh():
    return plsc.VectorSubcoreMesh(core_axis_name="c", subcore_axis_name="s",
                                  num_cores=NC, num_subcores=NS)


_SC_PARAMS = pltpu.CompilerParams(needs_layout_passes=False)


def _vfill(ref, nwords, value, dtype):
    vec = jnp.full((16,), value, dtype)

    def body(i, carry):
        ref[pl.ds(i * 16, 16)] = vec
        return carry

    lax.fori_loop(0, nwords // 16, body, 0)


def _vfill2d(ref, nrows, value, dtype):
    vec = jnp.full((16,), value, dtype)

    def body(r, carry):
        for kk in range(F // 16):
            ref[r, pl.ds(kk * 16, 16)] = vec
        return carry

    lax.fori_loop(0, nrows, body, 0)


def _sc_count(idx, SP, B):
    (M,) = idx.shape
    PW = M // NW
    NB = PW // B
    S16 = SP // 16
    DB = 2048 if S16 % 2048 == 0 else S16
    ND = S16 // DB
    B16 = ((B + 15) // 16) * 16

    @functools.partial(
        pl.kernel,
        out_type=jax.ShapeDtypeStruct((2 * SP,), jnp.float32),
        mesh=_mesh(),
        compiler_params=_SC_PARAMS,
        scratch_types=[
            pltpu.VMEM((B,), jnp.int32),
            pltpu.VMEM((B16,), jnp.float32),
            pltpu.VMEM((DB,), jnp.float32),
            pltpu.VMEM_SHARED((SP,), jnp.float32),
        ],
    )
    def k(idx_hbm, out_hbm, istage, ones_v, zbuf, acc):
        c = lax.axis_index("c")
        s = lax.axis_index("s")
        wid = c * NS + s
        _vfill(ones_v, B16, 1.0, jnp.float32)
        _vfill(zbuf, DB, 0.0, jnp.float32)
        for dd in range(ND):
            pltpu.sync_copy(zbuf, acc.at[pl.ds(s * S16 + dd * DB, DB)])
        plsc.subcore_barrier()

        def blk(b, carry):
            pltpu.sync_copy(idx_hbm.at[pl.ds(wid * PW + b * B, B)], istage)
            pltpu.sync_copy(ones_v.at[pl.ds(0, B)], acc.at[istage], add=True)
            return carry

        lax.fori_loop(0, NB, blk, 0)
        plsc.subcore_barrier()
        for dd in range(ND):
            pltpu.sync_copy(acc.at[pl.ds(s * S16 + dd * DB, DB)],
                            out_hbm.at[pl.ds(c * SP + s * S16 + dd * DB, DB)])

    return k(idx)


def _sc_segsum_nodes(table, src, dst, with_rows=False, linear=False):
    (M,) = dst.shape
    PW = M // NW
    B = 200
    NB = PW // B
    R16 = NP // 16

    out_type = [jax.ShapeDtypeStruct((2 * NP, F), jnp.float32)]
    if with_rows:
        out_type.append(jax.ShapeDtypeStruct((M, F), jnp.float32))

    scratch = [
        pltpu.VMEM((B,), jnp.int32),
        pltpu.VMEM((B,), jnp.int32),
        pltpu.VMEM((B, F), jnp.float32),
        pltpu.VMEM((128, F), jnp.float32),
        pltpu.SemaphoreType.DMA,
        pltpu.VMEM_SHARED((NP, F), jnp.float32),
    ]

    @functools.partial(pl.kernel, out_type=out_type, mesh=_mesh(),
                       compiler_params=_SC_PARAMS, scratch_types=scratch)
    def k(table_hbm, src_hbm, dst_hbm, *refs):
        if with_rows:
            out_hbm, rows_hbm = refs[0], refs[1]
            sstage, dstage, rows, zbuf, sem, acc = refs[2:]
        else:
            out_hbm = refs[0]
            sstage, dstage, rows, zbuf, sem, acc = refs[1:]
        c = lax.axis_index("c")
        s = lax.axis_index("s")
        wid = c * NS + s
        _vfill2d(zbuf, 128, 0.0, jnp.float32)
        for kk in range(R16 // 128):
            pltpu.sync_copy(zbuf, acc.at[pl.ds(s * R16 + kk * 128, 128)])
        plsc.subcore_barrier()

        def blk(b, carry):
            base = wid * PW + b * B
            pltpu.sync_copy(dst_hbm.at[pl.ds(base, B)], dstage)
            if linear:
                pltpu.sync_copy(table_hbm.at[pl.ds(base, B)], rows)
            else:
                pltpu.sync_copy(src_hbm.at[pl.ds(base, B)], sstage)
                pltpu.async_copy(table_hbm.at[sstage], rows, sem).wait()
            if with_rows:
                pltpu.sync_copy(rows, rows_hbm.at[pl.ds(base, B)])
            pltpu.sync_copy(rows, acc.at[dstage], add=True)
            return carry

        lax.fori_loop(0, NB, blk, 0)
        plsc.subcore_barrier()
        pltpu.sync_copy(acc.at[pl.ds(s * R16, R16)],
                        out_hbm.at[pl.ds(c * NP + s * R16, R16)])

    res = k(table, src, dst)
    if with_rows:
        return tuple(res)
    res = jax.tree.leaves(res)
    return (res[0],)


CH = 10000
NCH_SC = 8
NCH_TOT = NC * NCH_SC
ACC_ROWS = 10240
SB = 2000
GB = 128
FB = 2048
CAPR = 40960
BINSZ = NS * NCH_TOT * CAPR
BUFW = 4352


def _sc_bin_lg(lsrc, ldst):
    scratch = ([pltpu.VMEM((SB,), jnp.int32), pltpu.VMEM((SB,), jnp.int32)]
               + [pltpu.VMEM((BUFW,), jnp.int32) for _ in range(2 * NCH_SC)]
               + [pltpu.VMEM((16,), jnp.int32)])

    @functools.partial(
        pl.kernel,
        out_type=[jax.ShapeDtypeStruct((BINSZ,), jnp.int32),
                  jax.ShapeDtypeStruct((BINSZ,), jnp.int32),
                  jax.ShapeDtypeStruct((NC * NS * 16,), jnp.int32)],
        mesh=_mesh(),
        compiler_params=_SC_PARAMS,
        scratch_types=scratch,
    )
    def k(lsrc_hbm, ldst_hbm, bls_hbm, bld_hbm, cnt_hbm,
          ls_st, ld_st, *rest):
        bls = list(rest[:NCH_SC])
        bld = list(rest[NCH_SC:2 * NCH_SC])
        cvm = rest[2 * NCH_SC]
        c = lax.axis_index("c")
        s = lax.axis_index("s")
        PW = E_LG // NS
        dump_ls = jnp.zeros((16,), jnp.int32)
        dump_ld = jnp.full((16,), CH, jnp.int32)

        def flush_one(ci, off, fill):
            reg = (s * NCH_TOT + c * NCH_SC + ci) * CAPR
            pos = pl.multiple_of(reg + off, 256)
            pltpu.sync_copy(bls[ci].at[pl.ds(0, FB)],
                            bls_hbm.at[pl.ds(pos, FB)])
            pltpu.sync_copy(bld[ci].at[pl.ds(0, FB)],
                            bld_hbm.at[pl.ds(pos, FB)])

            def shift(kk, carry, ci=ci):
                t1 = bls[ci][pl.ds(FB + kk * 16, 16)]
                t2 = bld[ci][pl.ds(FB + kk * 16, 16)]
                bls[ci][pl.ds(kk * 16, 16)] = t1
                bld[ci][pl.ds(kk * 16, 16)] = t2
                return carry

            lax.fori_loop(0, (fill - FB + 15) // 16, shift, 0)
            return off + FB, fill - FB

        def sblk(b, carry):
            pltpu.sync_copy(lsrc_hbm.at[pl.ds(s * PW + b * SB, SB)], ls_st)
            pltpu.sync_copy(ldst_hbm.at[pl.ds(s * PW + b * SB, SB)], ld_st)

            def vb(v, carry):
                st = list(carry)
                lsv = ls_st[pl.ds(v * 16, 16)]
                ldv = ld_st[pl.ds(v * 16, 16)]
                chv = ldv // CH
                for ci in range(NCH_SC):
                    fill = st[ci]
                    chunk = c * NCH_SC + ci
                    m = chv == chunk
                    plsc.store_compressed(bls[ci].at[pl.ds(fill, 16)], lsv,
                                          mask=m)
                    plsc.store_compressed(bld[ci].at[pl.ds(fill, 16)],
                                          ldv - chunk * CH, mask=m)
                    st[ci] = fill + jnp.sum(jnp.where(m, 1, 0))
                return tuple(st)

            fills = lax.fori_loop(0, SB // 16, vb, carry[:NCH_SC])
            st = list(fills) + list(carry[NCH_SC:])
            for ci in range(NCH_SC):
                off, fill = st[NCH_SC + ci], st[ci]
                off, fill = lax.cond(
                    fill >= FB,
                    lambda o, f, ci=ci: flush_one(ci, o, f),
                    lambda o, f: (o, f), off, fill)
                st[NCH_SC + ci], st[ci] = off, fill
            return tuple(st)

        st = lax.fori_loop(0, PW // SB, sblk, (0,) * (2 * NCH_SC))

        cvec = jnp.zeros((16,), jnp.int32)
        lane = lax.iota(jnp.int32, 16)
        for ci in range(NCH_SC):
            off, fill = st[NCH_SC + ci], st[ci]
            pad = (-(off + fill)) % (2 * GB)
            for kk in range(2 * GB // 16):
                bls[ci][pl.ds(fill + kk * 16, 16)] = dump_ls
                bld[ci][pl.ds(fill + kk * 16, 16)] = dump_ld
            fill = fill + pad
            total = off + fill
            reg = (s * NCH_TOT + c * NCH_SC + ci) * CAPR

            def fblk(j, carry, ci=ci, reg=reg, off=off):
                pos = pl.multiple_of(reg + off + j * 256, 256)
                pltpu.sync_copy(bls[ci].at[pl.ds(j * 256, 256)],
                                bls_hbm.at[pl.ds(pos, 256)])
                pltpu.sync_copy(bld[ci].at[pl.ds(j * 256, 256)],
                                bld_hbm.at[pl.ds(pos, 256)])
                return carry

            lax.fori_loop(0, (fill + 255) // 256, fblk, 0)
            cvec = cvec + jnp.where(lane == c * NCH_SC + ci, total, 0)
        cvm[pl.ds(0, 16)] = cvec
        pltpu.sync_copy(cvm, cnt_hbm.at[pl.ds((c * NS + s) * 16, 16)])

    return k(lsrc, ldst)


def _sc_segsum_lg_stream(y, bls, bld, counts):
    NSTR = 2
    SGB = 128
    SBK = 4096
    scratch = ([pltpu.VMEM((SBK,), jnp.int32), pltpu.VMEM((SBK,), jnp.int32)]
               + [pltpu.VMEM((SGB,), jnp.int32) for _ in range(NSTR)]
               + [pltpu.VMEM((SGB, F), jnp.float32) for _ in range(NSTR)]
               + [pltpu.VMEM((32, F), jnp.float32),
                  pltpu.VMEM((16,), jnp.int32)]
               + [pltpu.SemaphoreType.DMA] * (2 * NSTR)
               + [pltpu.VMEM_SHARED((ACC_ROWS, F), jnp.float32)])

    @functools.partial(
        pl.kernel,
        out_type=jax.ShapeDtypeStruct((E, F), jnp.float32),
        mesh=_mesh(),
        compiler_params=_SC_PARAMS,
        scratch_types=scratch,
    )
    def k(y_hbm, bls_hbm, bld_hbm, cnt_hbm, out_hbm, *refs):
        lsbig, ldbig = refs[0], refs[1]
        ldv = list(refs[2:2 + NSTR])
        rowsv = list(refs[2 + NSTR:2 + 2 * NSTR])
        zbuf = refs[2 + 2 * NSTR]
        cvm = refs[3 + 2 * NSTR]
        semg = list(refs[4 + 2 * NSTR:4 + 3 * NSTR])
        sema = list(refs[4 + 3 * NSTR:4 + 4 * NSTR])
        acc = refs[-1]
        c = lax.axis_index("c")
        s = lax.axis_index("s")
        _vfill2d(zbuf, 32, 0.0, jnp.float32)
        pltpu.sync_copy(cnt_hbm.at[pl.ds((c * NS + s) * 16, 16)], cvm)
        cvec = cvm[pl.ds(0, 16)]
        lane = lax.iota(jnp.int32, 16)

        for ci in range(NCH_SC):
            chunk = c * NCH_SC + ci
            base_seg = chunk * CH
            reg = (s * NCH_TOT + chunk) * CAPR
            for kk in range(ACC_ROWS // NS // 32):
                pltpu.sync_copy(zbuf, acc.at[pl.ds(s * (ACC_ROWS // NS)
                                                   + kk * 32, 32)])
            plsc.subcore_barrier()
            cnt = jnp.sum(jnp.where(lane == chunk, cvec, 0))

            def sblk(sb, carry):
                sbase = pl.multiple_of(reg + sb * SBK, 256)
                pltpu.sync_copy(bls_hbm.at[pl.ds(sbase, SBK)], lsbig)
                pltpu.sync_copy(bld_hbm.at[pl.ds(sbase, SBK)], ldbig)
                npair = jnp.minimum(cnt - sb * SBK, SBK) // (NSTR * SGB)

                def pair(p, carry):
                    b0 = p * NSTR * SGB
                    gs = []
                    for q in range(NSTR):
                        gs.append(pltpu.async_copy(
                            y_hbm.at[lsbig.at[pl.ds(b0 + q * SGB, SGB)]],
                            rowsv[q], semg[q]))
                    aa = []
                    for q in range(NSTR):
                        for kk in range(SGB // 16):
                            ldv[q][pl.ds(kk * 16, 16)] = ldbig[
                                pl.ds(b0 + q * SGB + kk * 16, 16)]
                        gs[q].wait()
                        aa.append(pltpu.async_copy(rowsv[q], acc.at[ldv[q]],
                                                   sema[q], add=True))
                    for q in range(NSTR):
                        aa[q].wait()
                    return carry

                lax.fori_loop(0, npair, pair, 0)
                return carry

            lax.fori_loop(0, (cnt + SBK - 1) // SBK, sblk, 0)
            plsc.subcore_barrier()

            @pl.when(s < 10)
            def _():
                pltpu.sync_copy(acc.at[pl.ds(s * 1000, 1000)],
                                out_hbm.at[pl.ds(base_seg + s * 1000, 1000)])

            plsc.subcore_barrier()

    return k(y, bls, bld, counts)


def _dot(a, w):
    return lax.dot_general(a, w, (((1,), (0,)), ((), ())),
                           preferred_element_type=jnp.float32)


def _tc_sum2(a, b):
    nrows = a.shape[0]
    RB = 2000
    G = nrows // RB

    def body(a_ref, b_ref, o_ref):
        o_ref[...] = a_ref[...] + b_ref[...]

    rb = lambda i: (i, 0)
    return pl.pallas_call(
        body,
        grid=(G,),
        in_specs=[pl.BlockSpec((RB, F), rb), pl.BlockSpec((RB, F), rb)],
        out_specs=pl.BlockSpec((RB, F), rb),
        out_shape=jax.ShapeDtypeStruct((nrows, F), jnp.float32),
    )(a, b)


def _tc_affine5(x, dg0, dg1, a3, a4, a5, wx, wd, w3, w4, w5, bsum):
    nrows = x.shape[0]
    RB = 2000
    G = nrows // RB

    def body(x_ref, dg0_ref, dg1_ref, a3_ref, a4_ref, a5_ref,
             wx_ref, wd_ref, w3_ref, w4_ref, w5_ref, b_ref,
             out_ref, st_ref, acc):
        i = pl.program_id(0)
        x = x_ref[...]
        deg = dg0_ref[...] + dg1_ref[...]
        t = (_dot(x, wx_ref[...]) + _dot(x * deg, wd_ref[...])
             + _dot(a3_ref[...], w3_ref[...]) + _dot(a4_ref[...], w4_ref[...])
             + _dot(a5_ref[...], w5_ref[...]) + b_ref[...])
        lane = lax.broadcasted_iota(jnp.int32, t.shape, 1)
        t = jnp.where(lane >= F // 2, jnp.maximum(t, 0.0), t)
        out_ref[...] = t

        @pl.when(i == 0)
        def _():
            acc[...] = jnp.zeros_like(acc)

        acc[0:1, :] += jnp.sum(t, axis=0, keepdims=True)
        acc[1:2, :] += jnp.sum(t * t, axis=0, keepdims=True)
        st_ref[...] = acc[...]

    rb = lambda i: (i, 0)
    c0 = lambda i: (0, 0)
    wspec = pl.BlockSpec((F, F), c0)
    return pl.pallas_call(
        body,
        grid=(G,),
        in_specs=[pl.BlockSpec((RB, F), rb),
                  pl.BlockSpec((RB, 1), rb), pl.BlockSpec((RB, 1), rb),
                  pl.BlockSpec((RB, F), rb), pl.BlockSpec((RB, F), rb),
                  pl.BlockSpec((RB, F), rb),
                  wspec, wspec, wspec, wspec, wspec,
                  pl.BlockSpec((1, F), c0)],
        out_specs=[pl.BlockSpec((RB, F), rb), pl.BlockSpec((2, F), c0)],
        out_shape=[jax.ShapeDtypeStruct((nrows, F), jnp.float32),
                   jax.ShapeDtypeStruct((2, F), jnp.float32)],
        scratch_shapes=[pltpu.VMEM((2, F), jnp.float32)],
    )(x, dg0, dg1, a3, a4, a5, wx, wd, w3, w4, w5, bsum)


def _tc_norm(t, stats, scale, bias, wf=None, bf=None):
    nrows = t.shape[0]
    RB = 2000
    G = nrows // RB
    inv_n = 1.0 / nrows
    fuse = wf is not None

    def body(*refs):
        if fuse:
            t_ref, st_ref, sc_ref, bi_ref, wf_ref, bf_ref, o_ref = refs
        else:
            t_ref, st_ref, sc_ref, bi_ref, o_ref = refs
        st = st_ref[...]
        mean = st[0:1, :] * inv_n
        var = st[1:2, :] * inv_n - mean * mean
        inv = lax.rsqrt(var + EPS)
        y = (t_ref[...] - mean) * (inv * sc_ref[...]) + bi_ref[...]
        if fuse:
            o_ref[...] = _dot(y, wf_ref[...]) + bf_ref[...]
        else:
            o_ref[...] = y

    rb = lambda i: (i, 0)
    c0 = lambda i: (0, 0)
    in_specs = [pl.BlockSpec((RB, F), rb), pl.BlockSpec((2, F), c0),
                pl.BlockSpec((1, F), c0), pl.BlockSpec((1, F), c0)]
    args = [t, stats, scale, bias]
    if fuse:
        in_specs += [pl.BlockSpec((F, OUT_FEATS), c0),
                     pl.BlockSpec((1, OUT_FEATS), c0)]
        args += [wf, bf]
        out_spec = pl.BlockSpec((RB, OUT_FEATS), rb)
        out_shape = jax.ShapeDtypeStruct((nrows, OUT_FEATS), jnp.float32)
    else:
        out_spec = pl.BlockSpec((RB, F), rb)
        out_shape = jax.ShapeDtypeStruct((nrows, F), jnp.float32)
    return pl.pallas_call(
        body, grid=(G,), in_specs=in_specs, out_specs=out_spec,
        out_shape=out_shape,
    )(*args)


def _split_partials(p):
    return p[:N], p[NP:NP + N]


def kernel(h, lg_h, edge_index, lg_edge_index, params):
    src, dst = edge_index[0], edge_index[1]
    lsrc, ldst = lg_edge_index[0], lg_edge_index[1]

    cnt_g = _sc_count(dst, NP, 1000)
    cnt_lg = _sc_count(ldst, EP, 2000)
    dg0 = cnt_g[:N].reshape(N, 1)
    dg1 = cnt_g[NP:NP + N].reshape(N, 1)
    dl0 = cnt_lg[:E].reshape(E, 1)
    dl1 = cnt_lg[EP:EP + E].reshape(E, 1)

    p0, p1 = params['modules'][0], params['modules'][1]

    def wmat(p, name):
        return p[name][0]

    z1p, pmpd_x = _sc_segsum_nodes(h, src, dst, with_rows=True)
    z1 = _tc_sum2(*_split_partials(z1p))
    (z2p,) = _sc_segsum_nodes(z1, src, dst)
    z2 = _tc_sum2(*_split_partials(z2p))
    (pyp,) = _sc_segsum_nodes(lg_h, src, dst, linear=True)
    py = _tc_sum2(*_split_partials(pyp))
    t_x, st_x = _tc_affine5(
        h, dg0, dg1, z1, z2, py,
        wmat(p0, 'theta_x'), wmat(p0, 'theta_deg'),
        p0['theta_list'][0][0], p0['theta_list'][1][0], wmat(p0, 'theta_y'),
        (p0['theta_x'][1] + p0['theta_deg'][1] + p0['theta_y'][1]
         + p0['theta_list'][0][1] + p0['theta_list'][1][1]).reshape(1, F))
    xn = _tc_norm(t_x, st_x, p0['bn_x'][0].reshape(1, F),
                  p0['bn_x'][1].reshape(1, F))

    bls, bld, bcnt = _sc_bin_lg(lsrc, ldst)
    w1 = _sc_segsum_lg_stream(lg_h, bls, bld, bcnt)
    w2 = _sc_segsum_lg_stream(w1, bls, bld, bcnt)
    t_y, st_y = _tc_affine5(
        lg_h, dl0, dl1, w1, w2, pmpd_x,
        wmat(p0, 'gamma_y'), wmat(p0, 'gamma_deg'),
        p0['gamma_list'][0][0], p0['gamma_list'][1][0], wmat(p0, 'gamma_x'),
        (p0['gamma_y'][1] + p0['gamma_deg'][1] + p0['gamma_x'][1]
         + p0['gamma_list'][0][1] + p0['gamma_list'][1][1]).reshape(1, F))
    yn = _tc_norm(t_y, st_y, p0['bn_y'][0].reshape(1, F),
                  p0['bn_y'][1].reshape(1, F))

    (z1p2,) = _sc_segsum_nodes(xn, src, dst)
    z1_2 = _tc_sum2(*_split_partials(z1p2))
    (z2p2,) = _sc_segsum_nodes(z1_2, src, dst)
    z2_2 = _tc_sum2(*_split_partials(z2p2))
    (pyp2,) = _sc_segsum_nodes(yn, src, dst, linear=True)
    py2 = _tc_sum2(*_split_partials(pyp2))
    t2, st2 = _tc_affine5(
        xn, dg0, dg1, z1_2, z2_2, py2,
        wmat(p1, 'theta_x'), wmat(p1, 'theta_deg'),
        p1['theta_list'][0][0], p1['theta_list'][1][0], wmat(p1, 'theta_y'),
        (p1['theta_x'][1] + p1['theta_deg'][1] + p1['theta_y'][1]
         + p1['theta_list'][0][1] + p1['theta_list'][1][1]).reshape(1, F))
    wf, bf = params['linear']
    return _tc_norm(t2, st2, p1['bn_x'][0].reshape(1, F),
                    p1['bn_x'][1].reshape(1, F),
                    wf=wf, bf=bf.reshape(1, OUT_FEATS))
